# baseline (speedup 1.0000x reference)
import functools

import jax
import jax.numpy as jnp
from jax import lax
from jax.experimental import pallas as pl
from jax.experimental.pallas import tpu as pltpu
from jax.experimental.pallas import tpu_sc as plsc


def _tile_top2(st, t, base, neg_inf):
    w = st.shape[1]
    cv1 = jnp.full((8, w), neg_inf)
    ci1 = jnp.zeros((8, w), jnp.int32)
    cv2 = jnp.full((8, w), neg_inf)
    for c in range(t // 8):
        s = st[c * 8:(c + 1) * 8, :]
        gt = s > cv1
        cv2 = jnp.maximum(cv2, jnp.minimum(cv1, s))
        cv1 = jnp.maximum(cv1, s)
        ci1 = jnp.where(gt, c, ci1)
    sub8 = lax.broadcasted_iota(jnp.int32, (8, w), 0)
    g1 = ci1 * 8 + sub8
    t1 = jnp.max(cv1, axis=0, keepdims=True)
    trel = jnp.min(jnp.where(cv1 == t1, g1, t), axis=0, keepdims=True)
    rstar = jnp.bitwise_and(trel, 7)
    v2c = jnp.max(jnp.where(sub8 == rstar, neg_inf, cv1), axis=0,
                  keepdims=True)
    t2 = jnp.maximum(v2c, jnp.max(cv2, axis=0, keepdims=True))
    return t1, trel + base, t2


def _merge_top2(run, tile):
    v1, i1, v2 = run
    t1, tg, t2 = tile
    i1n = jnp.where(t1 > v1, tg, i1)
    v2n = jnp.maximum(jnp.minimum(v1, t1), jnp.maximum(v2, t2))
    v1n = jnp.maximum(v1, t1)
    return v1n, i1n, v2n


def _nn_body(d0_ref, d1_ref, m0_ref, m1_ref, s_ref, *, n, m, t, ratio2):
    d0 = d0_ref[0]
    d1 = d1_ref[0]
    nt = m // t
    neg_inf = jnp.float32(-jnp.inf)

    qrun = (jnp.full((1, n), neg_inf), jnp.zeros((1, n), jnp.int32),
            jnp.full((1, n), neg_inf))
    krun = (jnp.full((1, m), neg_inf), jnp.zeros((1, m), jnp.int32),
            jnp.full((1, m), neg_inf))
    for mt in range(nt):
        a = d1[:, mt * t:(mt + 1) * t]
        st1 = lax.dot_general(a, d0, (((0,), (0,)), ((), ())),
                              preferred_element_type=jnp.float32)
        qrun = _merge_top2(qrun, _tile_top2(st1, t, mt * t, neg_inf))
        b = d0[:, mt * t:(mt + 1) * t]
        st2 = lax.dot_general(b, d1, (((0,), (0,)), ((), ())),
                              preferred_element_type=jnp.float32)
        krun = _merge_top2(krun, _tile_top2(st2, t, mt * t, neg_inf))

    v1, i1, v2 = qrun
    maskq = (2.0 * (1.0 - v1)) <= ratio2 * (2.0 * (1.0 - v2))
    m0_ref[0] = jnp.where(maskq, i1, -1).astype(jnp.int32)
    s_ref[0] = jnp.where(maskq, (v1 + 1.0) / 2.0, 0.0).astype(jnp.float32)

    w1, j1, w2 = krun
    maskk = (2.0 * (1.0 - w1)) <= ratio2 * (2.0 * (1.0 - w2))
    m1_ref[0] = jnp.where(maskk, j1, -1).astype(jnp.int32)


def _mutual_sc(m0_flat, m1_flat, n):
    total = m0_flat.shape[0]
    nw = 32
    chunk = total // nw
    mesh = plsc.VectorSubcoreMesh(core_axis_name="c", subcore_axis_name="s")

    @functools.partial(
        pl.kernel, mesh=mesh,
        out_type=jax.ShapeDtypeStruct((total,), jnp.int32),
        scratch_types=[pltpu.VMEM((chunk,), jnp.int32),
                       pltpu.VMEM((chunk,), jnp.int32),
                       pltpu.VMEM((chunk,), jnp.int32),
                       pltpu.VMEM((chunk,), jnp.int32)],
    )
    def mc(m0_hbm, m1_hbm, out_hbm, m0_v, idx_v, g_v, out_v):
        wid = lax.axis_index("s") * 2 + lax.axis_index("c")
        base = wid * chunk
        bstart = (base // n) * n
        pltpu.sync_copy(m0_hbm.at[pl.ds(base, chunk)], m0_v)
        off = base - bstart
        zero16 = jnp.zeros((16,), jnp.int32)
        iota16 = lax.broadcasted_iota(jnp.int32, (16,), 0)
        for j in range(chunk // 16):
            idx = m0_v[pl.ds(j * 16, 16)]
            idx_v[pl.ds(j * 16, 16)] = jnp.maximum(idx, zero16) + bstart
        for p in range(chunk // 128):
            pltpu.sync_copy(m1_hbm.at[idx_v.at[pl.ds(p * 128, 128)]],
                            g_v.at[pl.ds(p * 128, 128)])
        for j in range(chunk // 16):
            idx = m0_v[pl.ds(j * 16, 16)]
            g = g_v[pl.ds(j * 16, 16)]
            iloc = iota16 + (off + j * 16)
            ok = (idx >= zero16) & (g == iloc)
            out_v[pl.ds(j * 16, 16)] = jnp.where(ok, idx, -1)
        pltpu.sync_copy(out_v, out_hbm.at[pl.ds(base, chunk)])

    return mc(m0_flat, m1_flat)


def kernel(descriptors0, descriptors1):
    b, d, n = descriptors0.shape
    m = descriptors1.shape[2]
    t = 256
    body = functools.partial(_nn_body, n=n, m=m, t=t,
                             ratio2=0.8 ** 2)
    m0p, m1p, scores3 = pl.pallas_call(
        body,
        grid=(b,),
        in_specs=[pl.BlockSpec((1, d, n), lambda i: (i, 0, 0)),
                  pl.BlockSpec((1, d, m), lambda i: (i, 0, 0))],
        out_specs=[pl.BlockSpec((1, 1, n), lambda i: (i, 0, 0)),
                   pl.BlockSpec((1, 1, m), lambda i: (i, 0, 0)),
                   pl.BlockSpec((1, 1, n), lambda i: (i, 0, 0))],
        out_shape=[jax.ShapeDtypeStruct((b, 1, n), jnp.int32),
                   jax.ShapeDtypeStruct((b, 1, m), jnp.int32),
                   jax.ShapeDtypeStruct((b, 1, n), jnp.float32)],
        compiler_params=pltpu.CompilerParams(
            dimension_semantics=("parallel",)),
    )(descriptors0, descriptors1)
    matches = _mutual_sc(m0p.reshape(b * n), m1p.reshape(b * m), n)
    return matches.reshape(b, n), scores3.reshape(b, n)


# speedup vs baseline: 1.1744x; 1.1744x over previous
import functools

import jax
import jax.numpy as jnp
from jax import lax
from jax.experimental import pallas as pl
from jax.experimental.pallas import tpu as pltpu
from jax.experimental.pallas import tpu_sc as plsc


def _tile_top2(st, t, base, neg_inf):
    w = st.shape[1]
    cv1 = jnp.full((8, w), neg_inf)
    ci1 = jnp.zeros((8, w), jnp.int32)
    cv2 = jnp.full((8, w), neg_inf)
    for c in range(t // 8):
        s = st[c * 8:(c + 1) * 8, :]
        gt = s > cv1
        cv2 = jnp.maximum(cv2, jnp.minimum(cv1, s))
        cv1 = jnp.maximum(cv1, s)
        ci1 = jnp.where(gt, c, ci1)
    sub8 = lax.broadcasted_iota(jnp.int32, (8, w), 0)
    g1 = ci1 * 8 + sub8
    t1 = jnp.max(cv1, axis=0, keepdims=True)
    trel = jnp.min(jnp.where(cv1 == t1, g1, t), axis=0, keepdims=True)
    rstar = jnp.bitwise_and(trel, 7)
    v2c = jnp.max(jnp.where(sub8 == rstar, neg_inf, cv1), axis=0,
                  keepdims=True)
    t2 = jnp.maximum(v2c, jnp.max(cv2, axis=0, keepdims=True))
    return t1, trel + base, t2


def _merge_top2(run, tile):
    v1, i1, v2 = run
    t1, tg, t2 = tile
    i1n = jnp.where(t1 > v1, tg, i1)
    v2n = jnp.maximum(jnp.minimum(v1, t1), jnp.maximum(v2, t2))
    v1n = jnp.maximum(v1, t1)
    return v1n, i1n, v2n


def _nn_body(d0_ref, d1_ref, m0_ref, m1_ref, s_ref, *, n, m, t, ratio2):
    d0 = d0_ref[0]
    d1 = d1_ref[0]
    nt = m // t
    neg_inf = jnp.float32(-jnp.inf)

    qrun = (jnp.full((1, n), neg_inf), jnp.zeros((1, n), jnp.int32),
            jnp.full((1, n), neg_inf))
    krun = (jnp.full((1, m), neg_inf), jnp.zeros((1, m), jnp.int32),
            jnp.full((1, m), neg_inf))
    for mt in range(nt):
        a = d1[:, mt * t:(mt + 1) * t]
        st1 = lax.dot_general(a, d0, (((0,), (0,)), ((), ())),
                              preferred_element_type=jnp.float32)
        qrun = _merge_top2(qrun, _tile_top2(st1, t, mt * t, neg_inf))
        b = d0[:, mt * t:(mt + 1) * t]
        st2 = lax.dot_general(b, d1, (((0,), (0,)), ((), ())),
                              preferred_element_type=jnp.float32)
        krun = _merge_top2(krun, _tile_top2(st2, t, mt * t, neg_inf))

    v1, i1, v2 = qrun
    maskq = (2.0 * (1.0 - v1)) <= ratio2 * (2.0 * (1.0 - v2))
    m0_ref[0] = jnp.where(maskq, i1, -1).astype(jnp.int32)
    s_ref[0] = jnp.where(maskq, (v1 + 1.0) / 2.0, 0.0).astype(jnp.float32)

    w1, j1, w2 = krun
    maskk = (2.0 * (1.0 - w1)) <= ratio2 * (2.0 * (1.0 - w2))
    m1_ref[0] = jnp.where(maskk, j1, -1).astype(jnp.int32)


def _mutual_sc(m0_flat, m1_flat, n):
    total = m0_flat.shape[0]
    nw = 32
    chunk = total // nw
    mesh = plsc.VectorSubcoreMesh(core_axis_name="c", subcore_axis_name="s")

    @functools.partial(
        pl.kernel, mesh=mesh,
        out_type=jax.ShapeDtypeStruct((total,), jnp.int32),
        scratch_types=[pltpu.VMEM((chunk,), jnp.int32),
                       pltpu.VMEM((chunk,), jnp.int32),
                       pltpu.VMEM((chunk,), jnp.int32),
                       pltpu.VMEM((chunk,), jnp.int32)],
    )
    def mc(m0_hbm, m1_hbm, out_hbm, m0_v, idx_v, g_v, out_v):
        wid = lax.axis_index("s") * 2 + lax.axis_index("c")
        base = wid * chunk
        bstart = (base // n) * n
        pltpu.sync_copy(m0_hbm.at[pl.ds(base, chunk)], m0_v)
        off = base - bstart
        zero16 = jnp.zeros((16,), jnp.int32)
        iota16 = lax.broadcasted_iota(jnp.int32, (16,), 0)
        for j in range(chunk // 16):
            idx = m0_v[pl.ds(j * 16, 16)]
            idx_v[pl.ds(j * 16, 16)] = jnp.maximum(idx, zero16) + bstart
        for p in range(chunk // 128):
            pltpu.sync_copy(m1_hbm.at[idx_v.at[pl.ds(p * 128, 128)]],
                            g_v.at[pl.ds(p * 128, 128)])
        for j in range(chunk // 16):
            idx = m0_v[pl.ds(j * 16, 16)]
            g = g_v[pl.ds(j * 16, 16)]
            iloc = iota16 + (off + j * 16)
            ok = (idx >= zero16) & (g == iloc)
            out_v[pl.ds(j * 16, 16)] = jnp.where(ok, idx, -1)
        pltpu.sync_copy(out_v, out_hbm.at[pl.ds(base, chunk)])

    return mc(m0_flat, m1_flat)


def kernel(descriptors0, descriptors1):
    b, d, n = descriptors0.shape
    m = descriptors1.shape[2]
    t = 2048
    body = functools.partial(_nn_body, n=n, m=m, t=t,
                             ratio2=0.8 ** 2)
    m0p, m1p, scores3 = pl.pallas_call(
        body,
        grid=(b,),
        in_specs=[pl.BlockSpec((1, d, n), lambda i: (i, 0, 0)),
                  pl.BlockSpec((1, d, m), lambda i: (i, 0, 0))],
        out_specs=[pl.BlockSpec((1, 1, n), lambda i: (i, 0, 0)),
                   pl.BlockSpec((1, 1, m), lambda i: (i, 0, 0)),
                   pl.BlockSpec((1, 1, n), lambda i: (i, 0, 0))],
        out_shape=[jax.ShapeDtypeStruct((b, 1, n), jnp.int32),
                   jax.ShapeDtypeStruct((b, 1, m), jnp.int32),
                   jax.ShapeDtypeStruct((b, 1, n), jnp.float32)],
        compiler_params=pltpu.CompilerParams(
            dimension_semantics=("parallel",)),
    )(descriptors0, descriptors1)
    matches = _mutual_sc(m0p.reshape(b * n), m1p.reshape(b * m), n)
    return matches.reshape(b, n), scores3.reshape(b, n)


# speedup vs baseline: 1.2119x; 1.0319x over previous
import functools

import jax
import jax.numpy as jnp
from jax import lax
from jax.experimental import pallas as pl
from jax.experimental.pallas import tpu as pltpu
from jax.experimental.pallas import tpu_sc as plsc


def _tile_top2(st, t, base, neg_inf):
    w = st.shape[1]
    cv1 = jnp.full((8, w), neg_inf)
    ci1 = jnp.zeros((8, w), jnp.int32)
    cv2 = jnp.full((8, w), neg_inf)
    for c in range(t // 8):
        s = st[c * 8:(c + 1) * 8, :]
        gt = s > cv1
        cv2 = jnp.maximum(cv2, jnp.minimum(cv1, s))
        cv1 = jnp.maximum(cv1, s)
        ci1 = jnp.where(gt, c, ci1)
    sub8 = lax.broadcasted_iota(jnp.int32, (8, w), 0)
    g1 = ci1 * 8 + sub8
    t1 = jnp.max(cv1, axis=0, keepdims=True)
    trel = jnp.min(jnp.where(cv1 == t1, g1, t), axis=0, keepdims=True)
    rstar = jnp.bitwise_and(trel, 7)
    v2c = jnp.max(jnp.where(sub8 == rstar, neg_inf, cv1), axis=0,
                  keepdims=True)
    t2 = jnp.maximum(v2c, jnp.max(cv2, axis=0, keepdims=True))
    return t1, trel + base, t2


def _merge_top2(run, tile):
    v1, i1, v2 = run
    t1, tg, t2 = tile
    i1n = jnp.where(t1 > v1, tg, i1)
    v2n = jnp.maximum(jnp.minimum(v1, t1), jnp.maximum(v2, t2))
    v1n = jnp.maximum(v1, t1)
    return v1n, i1n, v2n


def _nn_body(d0_ref, d1_ref, m0_ref, s_ref, *, n, m, t, ratio2):
    d0 = d0_ref[0]
    d1 = d1_ref[0]
    nt = m // t
    neg_inf = jnp.float32(-jnp.inf)

    qrun = (jnp.full((1, n), neg_inf), jnp.zeros((1, n), jnp.int32),
            jnp.full((1, n), neg_inf))
    krun = (jnp.full((1, m), neg_inf), jnp.zeros((1, m), jnp.int32),
            jnp.full((1, m), neg_inf))
    for mt in range(nt):
        a = d1[:, mt * t:(mt + 1) * t]
        st1 = lax.dot_general(a, d0, (((0,), (0,)), ((), ())),
                              preferred_element_type=jnp.float32)
        qrun = _merge_top2(qrun, _tile_top2(st1, t, mt * t, neg_inf))
        b = d0[:, mt * t:(mt + 1) * t]
        st2 = lax.dot_general(b, d1, (((0,), (0,)), ((), ())),
                              preferred_element_type=jnp.float32)
        krun = _merge_top2(krun, _tile_top2(st2, t, mt * t, neg_inf))

    v1, i1, v2 = qrun
    maskq = (2.0 * (1.0 - v1)) <= ratio2 * (2.0 * (1.0 - v2))
    m0 = jnp.where(maskq, i1, -1).astype(jnp.int32)
    s_ref[0] = jnp.where(maskq, (v1 + 1.0) / 2.0, 0.0).astype(jnp.float32)

    w1, j1, w2 = krun
    maskk = (2.0 * (1.0 - w1)) <= ratio2 * (2.0 * (1.0 - w2))
    m1 = jnp.where(maskk, j1, -1).astype(jnp.int32)

    lane_n = lax.broadcasted_iota(jnp.int32, (1, n), 1)
    lane_m = lax.broadcasted_iota(jnp.int32, (1, m), 1)
    ckey = (m1 * m + lane_m).reshape(m, 1)
    target = lane_n * m + m0
    ok = jnp.zeros((1, n), jnp.bool_)
    for mt in range(m // 256):
        ck = ckey[mt * 256:(mt + 1) * 256, :]
        ok = ok | jnp.any(ck == target, axis=0, keepdims=True)
    m0_ref[0] = jnp.where(ok, m0, -1)


def _mutual_sc(m0_flat, m1_flat, n):
    total = m0_flat.shape[0]
    nw = 32
    chunk = total // nw
    mesh = plsc.VectorSubcoreMesh(core_axis_name="c", subcore_axis_name="s")

    @functools.partial(
        pl.kernel, mesh=mesh,
        out_type=jax.ShapeDtypeStruct((total,), jnp.int32),
        scratch_types=[pltpu.VMEM((chunk,), jnp.int32),
                       pltpu.VMEM((chunk,), jnp.int32),
                       pltpu.VMEM((chunk,), jnp.int32),
                       pltpu.VMEM((chunk,), jnp.int32)],
    )
    def mc(m0_hbm, m1_hbm, out_hbm, m0_v, idx_v, g_v, out_v):
        wid = lax.axis_index("s") * 2 + lax.axis_index("c")
        base = wid * chunk
        bstart = (base // n) * n
        pltpu.sync_copy(m0_hbm.at[pl.ds(base, chunk)], m0_v)
        off = base - bstart
        zero16 = jnp.zeros((16,), jnp.int32)
        iota16 = lax.broadcasted_iota(jnp.int32, (16,), 0)
        for j in range(chunk // 16):
            idx = m0_v[pl.ds(j * 16, 16)]
            idx_v[pl.ds(j * 16, 16)] = jnp.maximum(idx, zero16) + bstart
        for p in range(chunk // 128):
            pltpu.sync_copy(m1_hbm.at[idx_v.at[pl.ds(p * 128, 128)]],
                            g_v.at[pl.ds(p * 128, 128)])
        for j in range(chunk // 16):
            idx = m0_v[pl.ds(j * 16, 16)]
            g = g_v[pl.ds(j * 16, 16)]
            iloc = iota16 + (off + j * 16)
            ok = (idx >= zero16) & (g == iloc)
            out_v[pl.ds(j * 16, 16)] = jnp.where(ok, idx, -1)
        pltpu.sync_copy(out_v, out_hbm.at[pl.ds(base, chunk)])

    return mc(m0_flat, m1_flat)


def kernel(descriptors0, descriptors1):
    b, d, n = descriptors0.shape
    m = descriptors1.shape[2]
    t = 2048
    body = functools.partial(_nn_body, n=n, m=m, t=t,
                             ratio2=0.8 ** 2)
    matches3, scores3 = pl.pallas_call(
        body,
        grid=(b,),
        in_specs=[pl.BlockSpec((1, d, n), lambda i: (i, 0, 0)),
                  pl.BlockSpec((1, d, m), lambda i: (i, 0, 0))],
        out_specs=[pl.BlockSpec((1, 1, n), lambda i: (i, 0, 0)),
                   pl.BlockSpec((1, 1, n), lambda i: (i, 0, 0))],
        out_shape=[jax.ShapeDtypeStruct((b, 1, n), jnp.int32),
                   jax.ShapeDtypeStruct((b, 1, n), jnp.float32)],
        compiler_params=pltpu.CompilerParams(
            dimension_semantics=("parallel",)),
    )(descriptors0, descriptors1)
    return matches3.reshape(b, n), scores3.reshape(b, n)
